# baseline (speedup 1.0000x reference)
import functools

import jax
import jax.numpy as jnp
from jax import lax
from jax.experimental import pallas as pl
from jax.experimental.pallas import tpu as pltpu
from jax.experimental.pallas import tpu_sc as plsc

_NSEG = 512
_D = 128
_N = 320000
_NC = 2
_NS = 16
_NW = _NC * _NS
_ROWS_W = _N // _NW
_BLK = 40
_NBLK = _ROWS_W // _BLK
_NBUF = 5
_L = 16
_G = _D // _L

_mesh = plsc.VectorSubcoreMesh(core_axis_name="c", subcore_axis_name="s")


@functools.partial(
    pl.kernel,
    out_type=jax.ShapeDtypeStruct((_NC, _NSEG, _D), jnp.float32),
    mesh=_mesh,
    scratch_types=[
        pltpu.VMEM((_NBLK, _BLK), jnp.int32),
        pltpu.VMEM((_NBUF, _BLK, _D), jnp.float32),
        pltpu.VMEM((_NSEG, _D), jnp.float32),
        pltpu.VMEM((_NSEG // 128, 128), jnp.int32),
        pltpu.VMEM_SHARED((_NSEG, _D), jnp.float32),
        [pltpu.SemaphoreType.DMA] * _NBUF,
    ],
)
def _seg_sum_sc(h_hbm, idx_hbm, zeros_hbm, out_hbm,
                idx_v, buf, priv, midx, acc, sems):
    cid = lax.axis_index("c")
    sid = lax.axis_index("s")
    wid = cid * _NS + sid
    base = wid * _ROWS_W

    stripe = _NSEG // _NS
    pltpu.sync_copy(zeros_hbm.at[pl.ds(sid * stripe, stripe)],
                    acc.at[pl.ds(sid * stripe, stripe)])

    pltpu.sync_copy(idx_hbm.at[wid], idx_v)

    for b in range(_NBUF):
        pltpu.async_copy(h_hbm.at[pl.ds(base + b * _BLK, _BLK)],
                         buf.at[b], sems[b])

    zvec = jnp.zeros((_L,), jnp.float32)

    def zero_body(i, carry):
        for c in range(_G):
            priv[i, pl.ds(c * _L, _L)] = zvec
        return carry

    lax.fori_loop(0, _NSEG, zero_body, 0)

    ivec = lax.iota(jnp.int32, _L)
    for m in range(_NSEG // 128):
        for g in range(128 // _L):
            midx[m, pl.ds(g * _L, _L)] = ivec + (m * 128 + g * _L)

    plsc.subcore_barrier()

    def flush(cur_seg, carried):
        for c in range(_G):
            priv[cur_seg, pl.ds(c * _L, _L)] = (
                priv[cur_seg, pl.ds(c * _L, _L)] + carried[c])

    def block(jj, b, cur_seg, carried):
        pltpu.make_async_copy(
            h_hbm.at[pl.ds(base, _BLK)], buf.at[b], sems[b]).wait()
        bref = buf.at[b]

        i0 = idx_v[jj, pl.ds(0, _L)]
        i2 = idx_v[jj, pl.ds(_BLK - _L, _L)]
        mn = i0[0]
        mx = i2[_L - 1]
        uniform = mn == mx
        start_new = jnp.logical_or(jnp.logical_not(uniform), mn != cur_seg)

        @pl.when(start_new)
        def _flush():
            flush(cur_seg, carried)

        carried = [jnp.where(start_new, zvec, carried[c]) for c in range(_G)]

        def row_body(i, s):
            r = i * 2
            s = [s[c] + bref[r, pl.ds(c * _L, _L)] for c in range(_G)]
            s = [s[c] + bref[r + 1, pl.ds(c * _L, _L)] for c in range(_G)]
            return s

        bsum = lax.fori_loop(0, _BLK // 2, row_body, carried)

        @pl.when(jnp.logical_not(uniform))
        def _stream_block():
            pltpu.sync_copy(bref, acc.at[idx_v.at[jj]], add=True)

        carried = [jnp.where(uniform, bsum[c], zvec) for c in range(_G)]
        cur_seg = jnp.where(uniform, mn, mx)

        @pl.when(jj + _NBUF < _NBLK)
        def _prefetch():
            pltpu.async_copy(
                h_hbm.at[pl.ds(base + (jj + _NBUF) * _BLK, _BLK)],
                buf.at[b], sems[b])

        return cur_seg, carried

    def body(i, carry):
        cur_seg = carry[0]
        carried = list(carry[1:])
        j = i * _NBUF
        for b in range(_NBUF):
            cur_seg, carried = block(j + b, b, cur_seg, carried)
        return (cur_seg, *carried)

    init = (jnp.int32(0),) + tuple(
        jnp.zeros((_L,), jnp.float32) for _ in range(_G))
    fin = lax.fori_loop(0, _NBLK // _NBUF, body, init)
    flush(fin[0], list(fin[1:]))

    lo_v = idx_v[0, pl.ds(0, _L)]
    hi_v = idx_v[_NBLK - 1, pl.ds(_BLK - _L, _L)]
    lo = lo_v[0]
    hi = hi_v[_L - 1]
    for m in range(_NSEG // 128):
        @pl.when(jnp.logical_and(hi >= m * 128, lo < (m + 1) * 128))
        def _merge():
            pltpu.sync_copy(priv.at[pl.ds(m * 128, 128)],
                            acc.at[midx.at[m]], add=True)

    plsc.subcore_barrier()

    @pl.when(sid == 0)
    def _writeback():
        pltpu.sync_copy(acc, out_hbm.at[cid])


def _sum_body(p_ref, o_ref):
    o_ref[...] = p_ref[0] + p_ref[1]


_sum_tc = pl.pallas_call(
    _sum_body,
    out_shape=jax.ShapeDtypeStruct((_NSEG, _D), jnp.float32),
)


def kernel(H, batch_idx):
    idx = batch_idx.astype(jnp.int32).reshape(_NW, _NBLK, _BLK)
    zeros = jnp.zeros((_NSEG, _D), jnp.float32)
    partials = _seg_sum_sc(H, idx, zeros)
    return _sum_tc(partials)


# speedup vs baseline: 1.0971x; 1.0971x over previous
import functools

import jax
import jax.numpy as jnp
from jax import lax
from jax.experimental import pallas as pl
from jax.experimental.pallas import tpu as pltpu
from jax.experimental.pallas import tpu_sc as plsc

_NSEG = 512
_D = 128
_N = 320000
_NC = 2
_NS = 16
_NW = _NC * _NS
_ROWS_W = _N // _NW
_FBLK = 80
_NFBLK = _ROWS_W // _FBLK
_NBUF = 5
_SUB = 40
_NSUB = _FBLK // _SUB
_NBLK = _ROWS_W // _SUB
_PRIV = 256
_L = 16
_G = _D // _L

_mesh = plsc.VectorSubcoreMesh(core_axis_name="c", subcore_axis_name="s")


@functools.partial(
    pl.kernel,
    out_type=jax.ShapeDtypeStruct((_NC, _NSEG, _D), jnp.float32),
    mesh=_mesh,
    scratch_types=[
        pltpu.VMEM((_NBLK, _SUB), jnp.int32),
        pltpu.VMEM((_NBUF, _FBLK, _D), jnp.float32),
        pltpu.VMEM((_PRIV, _D), jnp.float32),
        pltpu.VMEM((_PRIV // 128, 128), jnp.int32),
        pltpu.VMEM_SHARED((_NSEG, _D), jnp.float32),
        [pltpu.SemaphoreType.DMA] * _NBUF,
    ],
)
def _seg_sum_sc(h_hbm, idx_hbm, zeros_hbm, out_hbm,
                idx_v, buf, priv, midx, acc, sems):
    cid = lax.axis_index("c")
    sid = lax.axis_index("s")
    wid = cid * _NS + sid
    base = wid * _ROWS_W

    stripe = _NSEG // _NS
    pltpu.sync_copy(zeros_hbm.at[pl.ds(sid * stripe, stripe)],
                    acc.at[pl.ds(sid * stripe, stripe)])

    pltpu.sync_copy(idx_hbm.at[wid], idx_v)
    lo_v = idx_v[0, pl.ds(0, _L)]
    hi_v = idx_v[_NBLK - 1, pl.ds(_SUB - _L, _L)]
    lo = lo_v[0]
    hi = hi_v[_L - 1]

    for b in range(_NBUF):
        pltpu.async_copy(h_hbm.at[pl.ds(base + b * _FBLK, _FBLK)],
                         buf.at[b], sems[b])

    zvec = jnp.zeros((_L,), jnp.float32)

    def zero_body(i, carry):
        for c in range(_G):
            priv[i, pl.ds(c * _L, _L)] = zvec
        return carry

    lax.fori_loop(0, _PRIV, zero_body, 0)

    ivec = lax.iota(jnp.int32, _L)
    for m in range(_PRIV // 128):
        for g in range(128 // _L):
            midx[m, pl.ds(g * _L, _L)] = jnp.minimum(
                ivec + (m * 128 + g * _L) + lo, _NSEG - 1)

    plsc.subcore_barrier()

    def flush(cur_seg, carried):
        row = jnp.minimum(cur_seg - lo, _PRIV - 1)
        for c in range(_G):
            priv[row, pl.ds(c * _L, _L)] = (
                priv[row, pl.ds(c * _L, _L)] + carried[c])

    def sub_block(jj, bref, roff, cur_seg, carried):
        i0 = idx_v[jj, pl.ds(0, _L)]
        i2 = idx_v[jj, pl.ds(_SUB - _L, _L)]
        mn = i0[0]
        mx = i2[_L - 1]
        uniform = jnp.logical_and(mn == mx, mx - lo < _PRIV)
        start_new = jnp.logical_or(jnp.logical_not(uniform), mn != cur_seg)

        @pl.when(start_new)
        def _flush():
            flush(cur_seg, carried)

        carried = [jnp.where(start_new, zvec, carried[c]) for c in range(_G)]

        def row_body(i, s):
            r = roff + i * 2
            s = [s[c] + bref[r, pl.ds(c * _L, _L)] for c in range(_G)]
            s = [s[c] + bref[r + 1, pl.ds(c * _L, _L)] for c in range(_G)]
            return s

        bsum = lax.fori_loop(0, _SUB // 2, row_body, carried)

        @pl.when(jnp.logical_not(uniform))
        def _stream_block():
            pltpu.sync_copy(bref.at[pl.ds(roff, _SUB)],
                            acc.at[idx_v.at[jj]], add=True)

        carried = [jnp.where(uniform, bsum[c], zvec) for c in range(_G)]
        cur_seg = jnp.where(uniform, mn, mx)
        return cur_seg, carried

    def body(i, carry):
        cur_seg = carry[0]
        carried = list(carry[1:])
        f0 = i * _NBUF
        for b in range(_NBUF):
            f = f0 + b
            pltpu.make_async_copy(
                h_hbm.at[pl.ds(base, _FBLK)], buf.at[b], sems[b]).wait()
            for h in range(_NSUB):
                cur_seg, carried = sub_block(
                    f * _NSUB + h, buf.at[b], h * _SUB, cur_seg, carried)

            @pl.when(f + _NBUF < _NFBLK)
            def _prefetch():
                pltpu.async_copy(
                    h_hbm.at[pl.ds(base + (f + _NBUF) * _FBLK, _FBLK)],
                    buf.at[b], sems[b])
        return (cur_seg, *carried)

    init = (lo,) + tuple(jnp.zeros((_L,), jnp.float32) for _ in range(_G))
    fin = lax.fori_loop(0, _NFBLK // _NBUF, body, init)
    flush(fin[0], list(fin[1:]))

    for m in range(_PRIV // 128):
        @pl.when(hi - lo >= m * 128)
        def _merge():
            pltpu.sync_copy(priv.at[pl.ds(m * 128, 128)],
                            acc.at[midx.at[m]], add=True)

    plsc.subcore_barrier()

    @pl.when(sid == 0)
    def _writeback():
        pltpu.sync_copy(acc, out_hbm.at[cid])


def _sum_body(p_ref, o_ref):
    o_ref[...] = p_ref[0] + p_ref[1]


_sum_tc = pl.pallas_call(
    _sum_body,
    out_shape=jax.ShapeDtypeStruct((_NSEG, _D), jnp.float32),
)


def kernel(H, batch_idx):
    idx = batch_idx.astype(jnp.int32).reshape(_NW, _NBLK, _SUB)
    zeros = jnp.zeros((_NSEG, _D), jnp.float32)
    partials = _seg_sum_sc(H, idx, zeros)
    return _sum_tc(partials)
